# baseline (speedup 1.0000x reference)
import jax
import jax.numpy as jnp
from jax.experimental import pallas as pl
from jax.experimental.pallas import tpu as pltpu


def _body(f_ref, w1_ref, b1_ref, w2_ref, b2_ref, wh_ref, bh_ref,
          outc_ref, outr_ref, acc_ref):
    k = pl.program_id(0)
    nk = pl.num_programs(0)
    i = pl.program_id(1)
    bn = f_ref.shape[0]
    rows = pl.ds(i * bn, bn)

    part = jnp.dot(f_ref[...], w1_ref[...], preferred_element_type=jnp.float32)

    @pl.when(k == 0)
    def _init():
        acc_ref[rows, :] = part

    @pl.when(k != 0)
    def _accum():
        acc_ref[rows, :] += part

    @pl.when(k == nk - 1)
    def _finish():
        x = jnp.maximum(acc_ref[rows, :] + b1_ref[...], 0.0)
        x = jnp.dot(x, w2_ref[...], preferred_element_type=jnp.float32)
        x = jnp.maximum(x + b2_ref[...], 0.0)
        y = jnp.dot(x, wh_ref[...], preferred_element_type=jnp.float32)
        y = y + bh_ref[...]
        nc = outc_ref.shape[1]
        outc_ref[...] = y[:, :nc]
        outr_ref[...] = y[:, nc:]


def kernel(feature_vectors, W1, b1, W2, b2, Wc, bc, Wr, br):
    N, D = feature_vectors.shape
    H = W1.shape[1]
    NC = Wc.shape[1]
    NR = Wr.shape[1]

    BN = 200
    BK = 1792
    assert N % BN == 0 and D % BK == 0
    grid = (D // BK, N // BN)

    Wh = jnp.concatenate([Wc, Wr], axis=1)
    bh = jnp.concatenate([bc, br])[None, :]
    b1_2d = b1[None, :]
    b2_2d = b2[None, :]

    outc, outr = pl.pallas_call(
        _body,
        grid=grid,
        in_specs=[
            pl.BlockSpec((BN, BK), lambda k, i: (i, k)),
            pl.BlockSpec((BK, H), lambda k, i: (k, 0)),
            pl.BlockSpec((1, H), lambda k, i: (0, 0)),
            pl.BlockSpec((H, H), lambda k, i: (0, 0)),
            pl.BlockSpec((1, H), lambda k, i: (0, 0)),
            pl.BlockSpec((H, NC + NR), lambda k, i: (0, 0)),
            pl.BlockSpec((1, NC + NR), lambda k, i: (0, 0)),
        ],
        out_specs=[
            pl.BlockSpec((BN, NC), lambda k, i: (i, 0)),
            pl.BlockSpec((BN, NR), lambda k, i: (i, 0)),
        ],
        out_shape=[
            jax.ShapeDtypeStruct((N, NC), jnp.float32),
            jax.ShapeDtypeStruct((N, NR), jnp.float32),
        ],
        scratch_shapes=[pltpu.VMEM((N, H), jnp.float32)],
        compiler_params=pltpu.CompilerParams(
            dimension_semantics=("arbitrary", "arbitrary"),
        ),
    )(feature_vectors, W1, b1_2d, W2, b2_2d, Wh, bh)
    return outc, outr
